# Initial kernel scaffold; baseline (speedup 1.0000x reference)
import jax
import jax.numpy as jnp
from jax.experimental import pallas as pl


def kernel(x, edge_index_cites, edge_index_cited_by, W1_cites, b1_cites, W1_cited_by, b1_cited_by, W2_cites, b2_cites, W2_cited_by, b2_cited_by):
    raise NotImplementedError("write your pallas kernel here")


# speedup vs baseline: 2.9015x; 2.9015x over previous
import functools

import jax
import jax.numpy as jnp
from jax import lax
from jax.experimental import pallas as pl
from jax.experimental.pallas import tpu as pltpu
from jax.experimental.pallas import tpu_sc as plsc

N = 10000
E = 320000
D = 128

NC = 2
NS = 16
K = 128

R = ((E + K - 1) // K + NC * NS - 1) // (NC * NS) * (NC * NS)
EP = R * K
ROWS_PER_CORE = R // NC
STEPS = ROWS_PER_CORE // NS

NP = 10112
STRIPE = NP // NS

f32 = jnp.float32
i32 = jnp.int32

_sc_mesh = plsc.VectorSubcoreMesh(core_axis_name="c", subcore_axis_name="s")


DW = 128


@functools.partial(
    pl.kernel,
    out_type=jax.ShapeDtypeStruct((NC, 4, NP, DW), f32),
    mesh=_sc_mesh,
    scratch_types=[
        pltpu.VMEM((2, K), i32),
        pltpu.VMEM((K, DW), f32),
        pltpu.VMEM_SHARED((NP, DW), f32),
    ],
)
def _sc_degrees(srcc, dstc, srcr, dstr, zD, ones_hbm, dpart, idx_v, ones_v,
                deg_s):
    c = lax.axis_index("c")
    s = lax.axis_index("s")
    pltpu.sync_copy(ones_hbm, ones_v)

    streams = (srcc, dstc, srcr, dstr)
    for j in range(4):
        pltpu.sync_copy(zD.at[pl.ds(s * STRIPE, STRIPE)],
                        deg_s.at[pl.ds(s * STRIPE, STRIPE)])
        plsc.subcore_barrier()

        def body(k, _, j=j):
            r = c * ROWS_PER_CORE + k * NS + s
            off = pl.multiple_of(r * K, K)
            pltpu.sync_copy(streams[j].at[pl.ds(off, K)], idx_v.at[0])
            pltpu.sync_copy(ones_v, deg_s.at[idx_v.at[0]], add=True)
            return 0

        lax.fori_loop(0, STEPS, body, 0)
        plsc.subcore_barrier()
        pltpu.sync_copy(deg_s.at[pl.ds(s * STRIPE, STRIPE)],
                        dpart.at[c, j, pl.ds(s * STRIPE, STRIPE)])
        plsc.subcore_barrier()


@functools.partial(
    pl.kernel,
    out_type=[jax.ShapeDtypeStruct((NC, NP, D), f32),
              jax.ShapeDtypeStruct((NC, NP, D), f32)],
    mesh=_sc_mesh,
    scratch_types=[
        pltpu.VMEM((2, K), i32),
        pltpu.VMEM((K, D), f32),
        pltpu.VMEM_SHARED((NP, D), f32),
        pltpu.SemaphoreType.DMA,
    ],
)
def _sc_aggregate(tc, tr, srcc, dstc, srcr, dstr, zN, aggc, aggr, idx_v,
                  rows_v, agg_s, sem):
    c = lax.axis_index("c")
    s = lax.axis_index("s")

    for table, src, dst, out in ((tc, srcc, dstc, aggc),
                                 (tr, srcr, dstr, aggr)):
        pltpu.sync_copy(zN.at[pl.ds(s * STRIPE, STRIPE)],
                        agg_s.at[pl.ds(s * STRIPE, STRIPE)])
        plsc.subcore_barrier()

        def body(k, _):
            r = c * ROWS_PER_CORE + k * NS + s
            off = pl.multiple_of(r * K, K)
            pltpu.sync_copy(src.at[pl.ds(off, K)], idx_v.at[0])
            pltpu.sync_copy(dst.at[pl.ds(off, K)], idx_v.at[1])
            pltpu.async_copy(table.at[idx_v.at[0]], rows_v, sem).wait()
            pltpu.sync_copy(rows_v, agg_s.at[idx_v.at[1]], add=True)
            return 0

        lax.fori_loop(0, STEPS, body, 0)
        plsc.subcore_barrier()
        pltpu.sync_copy(agg_s.at[pl.ds(s * STRIPE, STRIPE)],
                        out.at[c, pl.ds(s * STRIPE, STRIPE)])
        plsc.subcore_barrier()


BM = 2000


def _scale_body(x_ref, d_ref, xc_ref, xr_ref):
    d = d_ref[...]
    ds_ = d[:, :4] + d[:, 4:]
    so_c = lax.rsqrt(jnp.maximum(ds_[:, 0:1], 1.0))
    so_r = lax.rsqrt(jnp.maximum(ds_[:, 2:3], 1.0))
    xv = x_ref[...]
    xc_ref[...] = xv * so_c
    xr_ref[...] = xv * so_r


def _tc_scale(x, dT):
    grid = (N // BM,)
    return pl.pallas_call(
        _scale_body,
        grid=grid,
        in_specs=[
            pl.BlockSpec((BM, D), lambda i: (i, 0)),
            pl.BlockSpec((BM, 8), lambda i: (i, 0)),
        ],
        out_specs=[
            pl.BlockSpec((BM, D), lambda i: (i, 0)),
            pl.BlockSpec((BM, D), lambda i: (i, 0)),
        ],
        out_shape=[jax.ShapeDtypeStruct((N, D), f32),
                   jax.ShapeDtypeStruct((N, D), f32)],
    )(x, dT)


def _dense_body(relu_and_rescale, ac_ref, ar_ref, d_ref, wc_ref, wr_ref,
                bc_ref, br_ref, *out_refs):
    ac = ac_ref[0] + ac_ref[1]
    ar = ar_ref[0] + ar_ref[1]
    d = d_ref[...]
    ds_ = d[:, :4] + d[:, 4:]
    si_c = lax.rsqrt(jnp.maximum(ds_[:, 1:2], 1.0))
    si_r = lax.rsqrt(jnp.maximum(ds_[:, 3:4], 1.0))
    h = (jnp.dot(ac * si_c, wc_ref[...], preferred_element_type=f32)
         + jnp.dot(ar * si_r, wr_ref[...], preferred_element_type=f32)
         + bc_ref[...] + br_ref[...])
    if relu_and_rescale:
        h = jnp.maximum(h, 0.0)
        so_c = lax.rsqrt(jnp.maximum(ds_[:, 0:1], 1.0))
        so_r = lax.rsqrt(jnp.maximum(ds_[:, 2:3], 1.0))
        out_refs[0][...] = h * so_c
        out_refs[1][...] = h * so_r
    else:
        out_refs[0][...] = h


def _tc_dense(aggc, aggr, dT, wc, wr, bc, br, relu_and_rescale):
    grid = (N // BM,)
    n_out = 2 if relu_and_rescale else 1
    return pl.pallas_call(
        functools.partial(_dense_body, relu_and_rescale),
        grid=grid,
        in_specs=[
            pl.BlockSpec((NC, BM, D), lambda i: (0, i, 0)),
            pl.BlockSpec((NC, BM, D), lambda i: (0, i, 0)),
            pl.BlockSpec((BM, 8), lambda i: (i, 0)),
            pl.BlockSpec((D, D), lambda i: (0, 0)),
            pl.BlockSpec((D, D), lambda i: (0, 0)),
            pl.BlockSpec((1, D), lambda i: (0, 0)),
            pl.BlockSpec((1, D), lambda i: (0, 0)),
        ],
        out_specs=[pl.BlockSpec((BM, D), lambda i: (i, 0))] * n_out,
        out_shape=[jax.ShapeDtypeStruct((N, D), f32)] * n_out,
    )(aggc, aggr, dT, wc, wr, bc, br)


def _pad_idx(a):
    pad = EP - E
    return jnp.concatenate([a, jnp.full((pad,), N, i32)])


def _pad_table(t):
    return jnp.concatenate([t, jnp.zeros((NP - N, D), f32)], axis=0)


def kernel(x, edge_index_cites, edge_index_cited_by, W1_cites, b1_cites,
           W1_cited_by, b1_cited_by, W2_cites, b2_cites, W2_cited_by,
           b2_cited_by):
    srcc = _pad_idx(edge_index_cites[0])
    dstc = _pad_idx(edge_index_cites[1])
    srcr = _pad_idx(edge_index_cited_by[0])
    dstr = _pad_idx(edge_index_cited_by[1])

    ones_col = jnp.ones((K, DW), f32)
    zN = jnp.zeros((NP, D), f32)

    dpart = _sc_degrees(srcc, dstc, srcr, dstr, zN, ones_col)
    dT = dpart[:, :, :N, 0].reshape(8, N).transpose(1, 0)

    b1c = b1_cites.reshape(1, D)
    b1r = b1_cited_by.reshape(1, D)
    b2c = b2_cites.reshape(1, D)
    b2r = b2_cited_by.reshape(1, D)

    xc, xr = _tc_scale(x, dT)
    aggc, aggr = _sc_aggregate(_pad_table(xc), _pad_table(xr),
                               srcc, dstc, srcr, dstr, zN)
    h1c, h1r = _tc_dense(aggc[:, :N], aggr[:, :N], dT, W1_cites, W1_cited_by,
                         b1c, b1r, True)
    aggc2, aggr2 = _sc_aggregate(_pad_table(h1c), _pad_table(h1r),
                                 srcc, dstc, srcr, dstr, zN)
    (h2,) = _tc_dense(aggc2[:, :N], aggr2[:, :N], dT, W2_cites, W2_cited_by,
                      b2c, b2r, False)
    return h2


# speedup vs baseline: 3.4461x; 1.1877x over previous
import functools

import jax
import jax.numpy as jnp
from jax import lax
from jax.experimental import pallas as pl
from jax.experimental.pallas import tpu as pltpu
from jax.experimental.pallas import tpu_sc as plsc

N = 10000
E = 320000
D = 128

NC = 2
NS = 16
K = 128

R = ((E + K - 1) // K + NC * NS - 1) // (NC * NS) * (NC * NS)
EP = R * K
ROWS_PER_CORE = R // NC
STEPS = ROWS_PER_CORE // NS

NP = 10112
STRIPE = NP // NS

f32 = jnp.float32
i32 = jnp.int32

_sc_mesh = plsc.VectorSubcoreMesh(core_axis_name="c", subcore_axis_name="s")


DW = 128


@functools.partial(
    pl.kernel,
    out_type=jax.ShapeDtypeStruct((NC, 4, NP, DW), f32),
    mesh=_sc_mesh,
    scratch_types=[
        pltpu.VMEM((2, K), i32),
        pltpu.VMEM((K, DW), f32),
        pltpu.VMEM_SHARED((NP, DW), f32),
    ],
)
def _sc_degrees(srcc, dstc, srcr, dstr, zD, ones_hbm, dpart, idx_v, ones_v,
                deg_s):
    c = lax.axis_index("c")
    s = lax.axis_index("s")
    pltpu.sync_copy(ones_hbm, ones_v)

    streams = (srcc, dstc, srcr, dstr)
    for j in range(4):
        pltpu.sync_copy(zD.at[pl.ds(s * STRIPE, STRIPE)],
                        deg_s.at[pl.ds(s * STRIPE, STRIPE)])
        plsc.subcore_barrier()

        def body(k, _, j=j):
            r = c * ROWS_PER_CORE + k * NS + s
            off = pl.multiple_of(r * K, K)
            pltpu.sync_copy(streams[j].at[pl.ds(off, K)], idx_v.at[0])
            pltpu.sync_copy(ones_v, deg_s.at[idx_v.at[0]], add=True)
            return 0

        lax.fori_loop(0, STEPS, body, 0)
        plsc.subcore_barrier()
        pltpu.sync_copy(deg_s.at[pl.ds(s * STRIPE, STRIPE)],
                        dpart.at[c, j, pl.ds(s * STRIPE, STRIPE)])
        plsc.subcore_barrier()


@functools.partial(
    pl.kernel,
    out_type=[jax.ShapeDtypeStruct((NC, NP, D), f32),
              jax.ShapeDtypeStruct((NC, NP, D), f32)],
    mesh=_sc_mesh,
    scratch_types=[
        pltpu.VMEM((2, 2, K), i32),
        pltpu.VMEM((2, K, D), f32),
        pltpu.VMEM_SHARED((NP, D), f32),
        pltpu.SemaphoreType.DMA,
        pltpu.SemaphoreType.DMA,
    ],
)
def _sc_aggregate(tc, tr, srcc, dstc, srcr, dstr, zN, aggc, aggr, idx_v,
                  rows_v, agg_s, sem0, sem1):
    c = lax.axis_index("c")
    s = lax.axis_index("s")
    sems = (sem0, sem1)

    for table, src, dst, out in ((tc, srcc, dstc, aggc),
                                 (tr, srcr, dstr, aggr)):
        def stage(step, buf, src=src, dst=dst):
            off = pl.multiple_of((c * ROWS_PER_CORE + step * NS + s) * K, K)
            pltpu.sync_copy(src.at[pl.ds(off, K)], idx_v.at[buf, 0])
            pltpu.sync_copy(dst.at[pl.ds(off, K)], idx_v.at[buf, 1])

        def gather(buf, table=table):
            pltpu.async_copy(table.at[idx_v.at[buf, 0]], rows_v.at[buf],
                             sems[buf])

        def wait(buf, table=table):
            pltpu.make_async_copy(table.at[idx_v.at[buf, 0]],
                                  rows_v.at[buf], sems[buf]).wait()

        def scatter(buf):
            pltpu.sync_copy(rows_v.at[buf], agg_s.at[idx_v.at[buf, 1]],
                            add=True)

        pltpu.sync_copy(zN.at[pl.ds(s * STRIPE, STRIPE)],
                        agg_s.at[pl.ds(s * STRIPE, STRIPE)])
        plsc.subcore_barrier()

        stage(0, 0)
        gather(0)

        def pair(p, _):
            stage(2 * p + 1, 1)
            gather(1)
            wait(0)
            scatter(0)
            stage(2 * p + 2, 0)
            gather(0)
            wait(1)
            scatter(1)
            return 0

        lax.fori_loop(0, (STEPS - 1) // 2, pair, 0)
        wait(0)
        scatter(0)
        plsc.subcore_barrier()
        pltpu.sync_copy(agg_s.at[pl.ds(s * STRIPE, STRIPE)],
                        out.at[c, pl.ds(s * STRIPE, STRIPE)])
        plsc.subcore_barrier()


BM = 2000


def _scale_body(x_ref, d_ref, xc_ref, xr_ref):
    d = d_ref[...]
    ds_ = d[:, :4] + d[:, 4:]
    so_c = lax.rsqrt(jnp.maximum(ds_[:, 0:1], 1.0))
    so_r = lax.rsqrt(jnp.maximum(ds_[:, 2:3], 1.0))
    xv = x_ref[...]
    xc_ref[...] = xv * so_c
    xr_ref[...] = xv * so_r


def _tc_scale(x, dT):
    grid = (N // BM,)
    return pl.pallas_call(
        _scale_body,
        grid=grid,
        in_specs=[
            pl.BlockSpec((BM, D), lambda i: (i, 0)),
            pl.BlockSpec((BM, 8), lambda i: (i, 0)),
        ],
        out_specs=[
            pl.BlockSpec((BM, D), lambda i: (i, 0)),
            pl.BlockSpec((BM, D), lambda i: (i, 0)),
        ],
        out_shape=[jax.ShapeDtypeStruct((N, D), f32),
                   jax.ShapeDtypeStruct((N, D), f32)],
    )(x, dT)


def _dense_body(relu_and_rescale, ac_ref, ar_ref, d_ref, wc_ref, wr_ref,
                bc_ref, br_ref, *out_refs):
    ac = ac_ref[0] + ac_ref[1]
    ar = ar_ref[0] + ar_ref[1]
    d = d_ref[...]
    ds_ = d[:, :4] + d[:, 4:]
    si_c = lax.rsqrt(jnp.maximum(ds_[:, 1:2], 1.0))
    si_r = lax.rsqrt(jnp.maximum(ds_[:, 3:4], 1.0))
    h = (jnp.dot(ac * si_c, wc_ref[...], preferred_element_type=f32)
         + jnp.dot(ar * si_r, wr_ref[...], preferred_element_type=f32)
         + bc_ref[...] + br_ref[...])
    if relu_and_rescale:
        h = jnp.maximum(h, 0.0)
        so_c = lax.rsqrt(jnp.maximum(ds_[:, 0:1], 1.0))
        so_r = lax.rsqrt(jnp.maximum(ds_[:, 2:3], 1.0))
        out_refs[0][...] = h * so_c
        out_refs[1][...] = h * so_r
    else:
        out_refs[0][...] = h


def _tc_dense(aggc, aggr, dT, wc, wr, bc, br, relu_and_rescale):
    grid = (N // BM,)
    n_out = 2 if relu_and_rescale else 1
    return pl.pallas_call(
        functools.partial(_dense_body, relu_and_rescale),
        grid=grid,
        in_specs=[
            pl.BlockSpec((NC, BM, D), lambda i: (0, i, 0)),
            pl.BlockSpec((NC, BM, D), lambda i: (0, i, 0)),
            pl.BlockSpec((BM, 8), lambda i: (i, 0)),
            pl.BlockSpec((D, D), lambda i: (0, 0)),
            pl.BlockSpec((D, D), lambda i: (0, 0)),
            pl.BlockSpec((1, D), lambda i: (0, 0)),
            pl.BlockSpec((1, D), lambda i: (0, 0)),
        ],
        out_specs=[pl.BlockSpec((BM, D), lambda i: (i, 0))] * n_out,
        out_shape=[jax.ShapeDtypeStruct((N, D), f32)] * n_out,
    )(aggc, aggr, dT, wc, wr, bc, br)


def _pad_idx(a):
    pad = EP - E
    return jnp.concatenate([a, jnp.full((pad,), N, i32)])


def _pad_table(t):
    return jnp.concatenate([t, jnp.zeros((NP - N, D), f32)], axis=0)


def kernel(x, edge_index_cites, edge_index_cited_by, W1_cites, b1_cites,
           W1_cited_by, b1_cited_by, W2_cites, b2_cites, W2_cited_by,
           b2_cited_by):
    srcc = _pad_idx(edge_index_cites[0])
    dstc = _pad_idx(edge_index_cites[1])
    srcr = _pad_idx(edge_index_cited_by[0])
    dstr = _pad_idx(edge_index_cited_by[1])

    ones_col = jnp.ones((K, DW), f32)
    zN = jnp.zeros((NP, D), f32)

    dpart = _sc_degrees(srcc, dstc, srcr, dstr, zN, ones_col)
    dT = dpart[:, :, :N, 0].reshape(8, N).transpose(1, 0)

    b1c = b1_cites.reshape(1, D)
    b1r = b1_cited_by.reshape(1, D)
    b2c = b2_cites.reshape(1, D)
    b2r = b2_cited_by.reshape(1, D)

    xc, xr = _tc_scale(x, dT)
    aggc, aggr = _sc_aggregate(_pad_table(xc), _pad_table(xr),
                               srcc, dstc, srcr, dstr, zN)
    h1c, h1r = _tc_dense(aggc[:, :N], aggr[:, :N], dT, W1_cites, W1_cited_by,
                         b1c, b1r, True)
    aggc2, aggr2 = _sc_aggregate(_pad_table(h1c), _pad_table(h1r),
                                 srcc, dstc, srcr, dstr, zN)
    (h2,) = _tc_dense(aggc2[:, :N], aggr2[:, :N], dT, W2_cites, W2_cited_by,
                      b2c, b2r, False)
    return h2


# speedup vs baseline: 3.4489x; 1.0008x over previous
import functools

import jax
import jax.numpy as jnp
from jax import lax
from jax.experimental import pallas as pl
from jax.experimental.pallas import tpu as pltpu
from jax.experimental.pallas import tpu_sc as plsc

N = 10000
E = 320000
D = 128

NC = 2
NS = 16
K = 128

R = ((E + K - 1) // K + NC * NS - 1) // (NC * NS) * (NC * NS)
EP = R * K
ROWS_PER_CORE = R // NC
STEPS = ROWS_PER_CORE // NS

NP = 10112
STRIPE = NP // NS

f32 = jnp.float32
i32 = jnp.int32

_sc_mesh = plsc.VectorSubcoreMesh(core_axis_name="c", subcore_axis_name="s")


@functools.partial(
    pl.kernel,
    out_type=jax.ShapeDtypeStruct((NC, NP, D), f32),
    mesh=_sc_mesh,
    scratch_types=[
        pltpu.VMEM((2, K), i32),
        pltpu.VMEM((K, D), f32),
        pltpu.VMEM_SHARED((NP, D), f32),
    ],
)
def _sc_degrees(srcc, dstc, srcr, dstr, zN, ones4, dpart, idx_v, ones_v,
                deg_s):
    c = lax.axis_index("c")
    s = lax.axis_index("s")

    pltpu.sync_copy(zN.at[pl.ds(s * STRIPE, STRIPE)],
                    deg_s.at[pl.ds(s * STRIPE, STRIPE)])
    plsc.subcore_barrier()

    for j, stream in enumerate((srcc, dstc, srcr, dstr)):
        pltpu.sync_copy(ones4.at[j], ones_v)

        def body(step, _, stream=stream):
            off = pl.multiple_of((c * ROWS_PER_CORE + step * NS + s) * K, K)
            pltpu.sync_copy(stream.at[pl.ds(off, K)], idx_v.at[0])
            pltpu.sync_copy(ones_v, deg_s.at[idx_v.at[0]], add=True)
            return 0

        lax.fori_loop(0, STEPS, body, 0)

    plsc.subcore_barrier()
    pltpu.sync_copy(deg_s.at[pl.ds(s * STRIPE, STRIPE)],
                    dpart.at[c, pl.ds(s * STRIPE, STRIPE)])


@functools.partial(
    pl.kernel,
    out_type=[jax.ShapeDtypeStruct((NC, NP, D), f32),
              jax.ShapeDtypeStruct((NC, NP, D), f32)],
    mesh=_sc_mesh,
    scratch_types=[
        pltpu.VMEM((2, 2, K), i32),
        pltpu.VMEM((2, K, D), f32),
        pltpu.VMEM_SHARED((NP, D), f32),
        pltpu.SemaphoreType.DMA,
        pltpu.SemaphoreType.DMA,
    ],
)
def _sc_aggregate(tc, tr, srcc, dstc, srcr, dstr, zN, aggc, aggr, idx_v,
                  rows_v, agg_s, sem0, sem1):
    c = lax.axis_index("c")
    s = lax.axis_index("s")
    sems = (sem0, sem1)

    for table, src, dst, out in ((tc, srcc, dstc, aggc),
                                 (tr, srcr, dstr, aggr)):
        def stage(step, buf, src=src, dst=dst):
            off = pl.multiple_of((c * ROWS_PER_CORE + step * NS + s) * K, K)
            pltpu.sync_copy(src.at[pl.ds(off, K)], idx_v.at[buf, 0])
            pltpu.sync_copy(dst.at[pl.ds(off, K)], idx_v.at[buf, 1])

        def gather(buf, table=table):
            pltpu.async_copy(table.at[idx_v.at[buf, 0]], rows_v.at[buf],
                             sems[buf])

        def wait(buf, table=table):
            pltpu.make_async_copy(table.at[idx_v.at[buf, 0]],
                                  rows_v.at[buf], sems[buf]).wait()

        def scatter(buf):
            pltpu.sync_copy(rows_v.at[buf], agg_s.at[idx_v.at[buf, 1]],
                            add=True)

        pltpu.sync_copy(zN.at[pl.ds(s * STRIPE, STRIPE)],
                        agg_s.at[pl.ds(s * STRIPE, STRIPE)])
        plsc.subcore_barrier()

        stage(0, 0)
        gather(0)

        def pair(p, _):
            stage(2 * p + 1, 1)
            gather(1)
            wait(0)
            scatter(0)
            stage(2 * p + 2, 0)
            gather(0)
            wait(1)
            scatter(1)
            return 0

        lax.fori_loop(0, (STEPS - 1) // 2, pair, 0)
        wait(0)
        scatter(0)
        plsc.subcore_barrier()
        pltpu.sync_copy(agg_s.at[pl.ds(s * STRIPE, STRIPE)],
                        out.at[c, pl.ds(s * STRIPE, STRIPE)])
        plsc.subcore_barrier()


BM = 2000


def _scale_body(x_ref, d_ref, xc_ref, xr_ref):
    d = d_ref[...]
    ds_ = d[:, :4] + d[:, 4:]
    so_c = lax.rsqrt(jnp.maximum(ds_[:, 0:1], 1.0))
    so_r = lax.rsqrt(jnp.maximum(ds_[:, 2:3], 1.0))
    xv = x_ref[...]
    xc_ref[...] = xv * so_c
    xr_ref[...] = xv * so_r


def _tc_scale(x, dT):
    grid = (N // BM,)
    return pl.pallas_call(
        _scale_body,
        grid=grid,
        in_specs=[
            pl.BlockSpec((BM, D), lambda i: (i, 0)),
            pl.BlockSpec((BM, 8), lambda i: (i, 0)),
        ],
        out_specs=[
            pl.BlockSpec((BM, D), lambda i: (i, 0)),
            pl.BlockSpec((BM, D), lambda i: (i, 0)),
        ],
        out_shape=[jax.ShapeDtypeStruct((N, D), f32),
                   jax.ShapeDtypeStruct((N, D), f32)],
    )(x, dT)


def _dense_body(relu_and_rescale, ac_ref, ar_ref, d_ref, wc_ref, wr_ref,
                bc_ref, br_ref, *out_refs):
    ac = ac_ref[0] + ac_ref[1]
    ar = ar_ref[0] + ar_ref[1]
    d = d_ref[...]
    ds_ = d[:, :4] + d[:, 4:]
    si_c = lax.rsqrt(jnp.maximum(ds_[:, 1:2], 1.0))
    si_r = lax.rsqrt(jnp.maximum(ds_[:, 3:4], 1.0))
    h = (jnp.dot(ac * si_c, wc_ref[...], preferred_element_type=f32)
         + jnp.dot(ar * si_r, wr_ref[...], preferred_element_type=f32)
         + bc_ref[...] + br_ref[...])
    if relu_and_rescale:
        h = jnp.maximum(h, 0.0)
        so_c = lax.rsqrt(jnp.maximum(ds_[:, 0:1], 1.0))
        so_r = lax.rsqrt(jnp.maximum(ds_[:, 2:3], 1.0))
        out_refs[0][...] = h * so_c
        out_refs[1][...] = h * so_r
    else:
        out_refs[0][...] = h


def _tc_dense(aggc, aggr, dT, wc, wr, bc, br, relu_and_rescale):
    grid = (N // BM,)
    n_out = 2 if relu_and_rescale else 1
    return pl.pallas_call(
        functools.partial(_dense_body, relu_and_rescale),
        grid=grid,
        in_specs=[
            pl.BlockSpec((NC, BM, D), lambda i: (0, i, 0)),
            pl.BlockSpec((NC, BM, D), lambda i: (0, i, 0)),
            pl.BlockSpec((BM, 8), lambda i: (i, 0)),
            pl.BlockSpec((D, D), lambda i: (0, 0)),
            pl.BlockSpec((D, D), lambda i: (0, 0)),
            pl.BlockSpec((1, D), lambda i: (0, 0)),
            pl.BlockSpec((1, D), lambda i: (0, 0)),
        ],
        out_specs=[pl.BlockSpec((BM, D), lambda i: (i, 0))] * n_out,
        out_shape=[jax.ShapeDtypeStruct((N, D), f32)] * n_out,
    )(aggc, aggr, dT, wc, wr, bc, br)


def _pad_idx(a):
    pad = EP - E
    return jnp.concatenate([a, jnp.full((pad,), N, i32)])


def _pad_table(t):
    return jnp.concatenate([t, jnp.zeros((NP - N, D), f32)], axis=0)


def kernel(x, edge_index_cites, edge_index_cited_by, W1_cites, b1_cites,
           W1_cited_by, b1_cited_by, W2_cites, b2_cites, W2_cited_by,
           b2_cited_by):
    srcc = _pad_idx(edge_index_cites[0])
    dstc = _pad_idx(edge_index_cites[1])
    srcr = _pad_idx(edge_index_cited_by[0])
    dstr = _pad_idx(edge_index_cited_by[1])

    zN = jnp.zeros((NP, D), f32)
    lane_group = jnp.arange(D, dtype=i32) // 32
    ones4 = jnp.broadcast_to(
        (lane_group[None, None, :] == jnp.arange(4, dtype=i32)[:, None, None])
        .astype(f32), (4, K, D))

    dpart = _sc_degrees(srcc, dstc, srcr, dstr, zN, ones4)
    lanes4 = dpart[:, :N, ::32]
    dT = jnp.concatenate([lanes4[0], lanes4[1]], axis=1)

    b1c = b1_cites.reshape(1, D)
    b1r = b1_cited_by.reshape(1, D)
    b2c = b2_cites.reshape(1, D)
    b2r = b2_cited_by.reshape(1, D)

    xc, xr = _tc_scale(x, dT)
    aggc, aggr = _sc_aggregate(_pad_table(xc), _pad_table(xr),
                               srcc, dstc, srcr, dstr, zN)
    h1c, h1r = _tc_dense(aggc[:, :N], aggr[:, :N], dT, W1_cites, W1_cited_by,
                         b1c, b1r, True)
    aggc2, aggr2 = _sc_aggregate(_pad_table(h1c), _pad_table(h1r),
                                 srcc, dstc, srcr, dstr, zN)
    (h2,) = _tc_dense(aggc2[:, :N], aggr2[:, :N], dT, W2_cites, W2_cited_by,
                      b2c, b2r, False)
    return h2
